# Initial kernel scaffold; baseline (speedup 1.0000x reference)
import jax
import jax.numpy as jnp
from jax.experimental import pallas as pl


def kernel(x, edge_index, W1, b1, W2, b2):
    raise NotImplementedError("write your pallas kernel here")


# speedup vs baseline: 13.8653x; 13.8653x over previous
import functools

import jax
import jax.numpy as jnp
from jax import lax
from jax.experimental import pallas as pl
from jax.experimental.pallas import tpu as pltpu
from jax.experimental.pallas import tpu_sc as plsc

NC = 2
NS = 16
NW = NC * NS
GRP = 128


def _mesh():
    return plsc.VectorSubcoreMesh(
        core_axis_name="c", subcore_axis_name="s", num_cores=NC, num_subcores=NS
    )


def _make_deg_kernel(G, NP):
    RPT = NP // NS

    @functools.partial(
        pl.kernel,
        out_type=jax.ShapeDtypeStruct((NC, NP), jnp.float32),
        mesh=_mesh(),
        scratch_types=[
            pltpu.VMEM((G, GRP), jnp.int32),
            pltpu.VMEM((GRP,), jnp.float32),
            pltpu.VMEM((RPT,), jnp.float32),
            pltpu.VMEM_SHARED((NP,), jnp.float32),
        ],
    )
    def deg_kernel(dst_hbm, out_hbm, idx_v, ones_v, zbuf_v, acc):
        c = lax.axis_index("c")
        s = lax.axis_index("s")
        w = s * NC + c

        def fill_ones(i, _):
            ones_v[pl.ds(i * 16, 16)] = jnp.full((16,), 1.0, jnp.float32)
            return 0

        lax.fori_loop(0, GRP // 16, fill_ones, 0)

        def fill_zeros(i, _):
            zbuf_v[pl.ds(i * 16, 16)] = jnp.zeros((16,), jnp.float32)
            return 0

        lax.fori_loop(0, RPT // 16, fill_zeros, 0)
        pltpu.sync_copy(zbuf_v, acc.at[pl.ds(s * RPT, RPT)])
        plsc.subcore_barrier()

        pltpu.sync_copy(dst_hbm.at[w], idx_v)

        def body(g, _):
            pltpu.sync_copy(ones_v, acc.at[idx_v.at[g]], add=True)
            return 0

        lax.fori_loop(0, G, body, 0)
        plsc.subcore_barrier()
        pltpu.sync_copy(
            acc.at[pl.ds(s * RPT, RPT)], out_hbm.at[c].at[pl.ds(s * RPT, RPT)]
        )

    return deg_kernel


def _make_prop_kernel(G, NP, C):
    RPT = NP // NS
    ZR = 64

    @functools.partial(
        pl.kernel,
        out_type=jax.ShapeDtypeStruct((NC, NP, C), jnp.float32),
        mesh=_mesh(),
        scratch_types=[
            pltpu.VMEM((G, GRP), jnp.int32),
            pltpu.VMEM((G, GRP), jnp.int32),
            pltpu.VMEM((GRP, C), jnp.float32),
            pltpu.VMEM((ZR, C), jnp.float32),
            pltpu.VMEM_SHARED((NP, C), jnp.float32),
            pltpu.SemaphoreType.DMA,
        ],
    )
    def prop_kernel(tab_hbm, src_hbm, dst_hbm, out_hbm,
                    src_v, dst_v, rows_v, zbuf_v, acc, sem):
        c = lax.axis_index("c")
        s = lax.axis_index("s")
        w = s * NC + c
        CV = C // 16

        def fill_zeros(i, _):
            r = i // CV
            k = i % CV
            zbuf_v[r, pl.ds(k * 16, 16)] = jnp.zeros((16,), jnp.float32)
            return 0

        lax.fori_loop(0, ZR * CV, fill_zeros, 0)

        def zero_acc(i, _):
            pltpu.sync_copy(zbuf_v, acc.at[pl.ds(s * RPT + i * ZR, ZR)])
            return 0

        lax.fori_loop(0, RPT // ZR, zero_acc, 0)
        plsc.subcore_barrier()

        pltpu.sync_copy(src_hbm.at[w], src_v)
        pltpu.sync_copy(dst_hbm.at[w], dst_v)

        def body(g, _):
            pltpu.async_copy(tab_hbm.at[src_v.at[g]], rows_v, sem).wait()
            pltpu.sync_copy(rows_v, acc.at[dst_v.at[g]], add=True)
            return 0

        lax.fori_loop(0, G, body, 0)
        plsc.subcore_barrier()
        pltpu.sync_copy(
            acc.at[pl.ds(s * RPT, RPT)], out_hbm.at[c].at[pl.ds(s * RPT, RPT)]
        )

    return prop_kernel


def _stage1(x, W1, degp, N, C1):
    def body(x_ref, w_ref, degp_ref, h_ref, dinv_ref):
        deg = degp_ref[0, :N] + degp_ref[1, :N] + 1.0
        dinv = lax.rsqrt(deg)
        h = jnp.dot(x_ref[...], w_ref[...], preferred_element_type=jnp.float32)
        h_ref[...] = h * dinv[:, None]
        dinv_ref[...] = dinv

    return pl.pallas_call(
        body,
        out_shape=[
            jax.ShapeDtypeStruct((N, C1), jnp.float32),
            jax.ShapeDtypeStruct((N,), jnp.float32),
        ],
    )(x, W1, degp)


def _stage2(p1, h1p, dinv, b1, N, C1):
    def body(p_ref, h1_ref, dinv_ref, b_ref, out_ref):
        pp = p_ref[0, :N, :] + p_ref[1, :N, :]
        dinv = dinv_ref[...]
        hid = dinv[:, None] * (pp + h1_ref[...]) + b_ref[...][None, :]
        out_ref[...] = jnp.maximum(hid, 0.0) * dinv[:, None]

    return pl.pallas_call(
        body,
        out_shape=jax.ShapeDtypeStruct((N, C1), jnp.float32),
    )(p1, h1p, dinv, b1)


def _stage3(p2, h2p, dinv, W2, b2, N, C2):
    def body(p_ref, h2_ref, dinv_ref, w_ref, b_ref, out_ref):
        pp = p_ref[0, :N, :] + p_ref[1, :N, :]
        z = dinv_ref[...][:, None] * (pp + h2_ref[...])
        o = jnp.dot(z, w_ref[...], preferred_element_type=jnp.float32)
        o = o + b_ref[...][None, :]
        m = jnp.max(o, axis=1, keepdims=True)
        lse = jnp.log(jnp.sum(jnp.exp(o - m), axis=1, keepdims=True))
        out_ref[...] = o - m - lse

    return pl.pallas_call(
        body,
        out_shape=jax.ShapeDtypeStruct((N, C2), jnp.float32),
    )(p2, h2p, dinv, W2, b2)


def kernel(x, edge_index, W1, b1, W2, b2):
    N, C1 = x.shape
    C2 = W2.shape[1]
    E = edge_index.shape[1]

    G = -(-E // (NW * GRP))
    EP = NW * G * GRP
    NP = -(-(N + 1) // (NS * 16)) * (NS * 16)

    pad = EP - E
    src = jnp.concatenate([edge_index[0], jnp.zeros((pad,), jnp.int32)])
    dst = jnp.concatenate([edge_index[1], jnp.full((pad,), N, jnp.int32)])
    srcp = src.reshape(NW, G, GRP)
    dstp = dst.reshape(NW, G, GRP)

    prop = _make_prop_kernel(G, NP, C1)
    degp = _make_deg_kernel(G, NP)(dstp)
    h1p, dinv = _stage1(x, W1, degp, N, C1)
    p1 = prop(h1p, srcp, dstp)
    h2p = _stage2(p1, h1p, dinv, b1, N, C1)
    p2 = prop(h2p, srcp, dstp)
    return _stage3(p2, h2p, dinv, W2, b2, N, C2)
